# Initial kernel scaffold; baseline (speedup 1.0000x reference)
import jax
import jax.numpy as jnp
from jax.experimental import pallas as pl


def kernel(input_ids, table):
    raise NotImplementedError("write your pallas kernel here")


# speedup vs baseline: 1.0199x; 1.0199x over previous
import functools

import jax
import jax.numpy as jnp
from jax import lax
from jax.experimental import pallas as pl
from jax.experimental.pallas import tpu as pltpu
from jax.experimental.pallas import tpu_sc as plsc

_CHUNK = 8
_NBUF = 4
_LOOKAHEAD = 2


def _make_gather(n_rows, hidden):
    info = plsc.get_sparse_core_info()
    nc, ns = info.num_cores, info.num_subcores
    nw = nc * ns
    assert n_rows % (nw * _CHUNK) == 0
    per_w = n_rows // nw
    n_chunks = per_w // _CHUNK
    assert n_chunks % _NBUF == 0 and n_chunks // _NBUF >= 2

    mesh = plsc.VectorSubcoreMesh(core_axis_name="c", subcore_axis_name="s")

    @functools.partial(
        pl.kernel,
        out_type=jax.ShapeDtypeStruct((n_rows, hidden), jnp.float32),
        mesh=mesh,
        scratch_types=(
            [pltpu.VMEM((per_w,), jnp.int32)]
            + [pltpu.VMEM((_CHUNK, hidden), jnp.float32) for _ in range(_NBUF)]
            + [pltpu.SemaphoreType.DMA for _ in range(2 * _NBUF)]
        ),
    )
    def gather_kernel(table, ids, out, idx_v, *rest):
        rows = rest[:_NBUF]
        in_sems = rest[_NBUF:2 * _NBUF]
        out_sems = rest[2 * _NBUF:]

        wid = lax.axis_index("s") * nc + lax.axis_index("c")
        base = wid * per_w
        pltpu.sync_copy(ids.at[pl.ds(base, per_w)], idx_v)

        def gather_desc(g, b):
            return pltpu.make_async_copy(
                table.at[idx_v.at[pl.ds(g * _CHUNK, _CHUNK)]],
                rows[b],
                in_sems[b],
            )

        def out_desc(g, b):
            return pltpu.make_async_copy(
                rows[b],
                out.at[pl.ds(base + g * _CHUNK, _CHUNK)],
                out_sems[b],
            )

        def visit(g, b, skip_out_wait=False, skip_gather=False):
            bn = (b + _LOOKAHEAD) % _NBUF
            if not skip_out_wait:
                out_desc(g + _LOOKAHEAD - _NBUF, bn).wait()
            if not skip_gather:
                gather_desc(g + _LOOKAHEAD, bn).start()
            gather_desc(g, b).wait()
            out_desc(g, b).start()

        for g in range(_LOOKAHEAD):
            gather_desc(g, g).start()
        for b in range(_NBUF):
            visit(b, b, skip_out_wait=(b + _LOOKAHEAD < _NBUF))

        def body(o, _):
            g0 = o * _NBUF
            for b in range(_NBUF):
                visit(g0 + b, b)
            return 0

        lax.fori_loop(1, n_chunks // _NBUF - 1, body, 0)

        g0 = n_chunks - _NBUF
        for b in range(_NBUF):
            visit(g0 + b, b, skip_gather=(b >= _NBUF - _LOOKAHEAD))
        for b in range(_NBUF - _LOOKAHEAD, _NBUF):
            out_desc(g0 + b, b).wait()

    return gather_kernel


def kernel(input_ids, table):
    b, s = input_ids.shape
    _, hidden = table.shape
    ids_t = input_ids.T.reshape(-1).astype(jnp.int32)
    flat = _make_gather(b * s, hidden)(table, ids_t)
    return flat.reshape(s, b, hidden)


# speedup vs baseline: 2.3795x; 2.3331x over previous
import functools

import jax
import jax.numpy as jnp
from jax import lax
from jax.experimental import pallas as pl
from jax.experimental.pallas import tpu as pltpu
from jax.experimental.pallas import tpu_sc as plsc

_CHUNK = 8
_NBUF = 4
_LOOKAHEAD = 2


def _make_gather(seq, batch, hidden):
    n_rows = seq * batch
    info = plsc.get_sparse_core_info()
    nc, ns = info.num_cores, info.num_subcores
    nw = nc * ns
    assert n_rows % (nw * _CHUNK) == 0
    per_w = n_rows // nw
    n_chunks = per_w // _CHUNK
    assert n_chunks % _NBUF == 0 and n_chunks // _NBUF >= 2

    mesh = plsc.VectorSubcoreMesh(core_axis_name="c", subcore_axis_name="s")

    @functools.partial(
        pl.kernel,
        out_type=jax.ShapeDtypeStruct((seq, batch, hidden), jnp.float32),
        mesh=mesh,
        scratch_types=(
            [pltpu.VMEM((per_w,), jnp.int32)]
            + [pltpu.VMEM((_CHUNK, hidden), jnp.float32) for _ in range(_NBUF)]
            + [pltpu.SemaphoreType.DMA for _ in range(2 * _NBUF)]
        ),
    )
    def gather_kernel(table, ids, out3d, idx_v, *rest):
        out = out3d.reshape(n_rows, hidden)
        rows = rest[:_NBUF]
        in_sems = rest[_NBUF:2 * _NBUF]
        out_sems = rest[2 * _NBUF:]

        wid = lax.axis_index("s") * nc + lax.axis_index("c")
        base = wid * per_w
        pltpu.sync_copy(ids.at[pl.ds(base, per_w)], idx_v)

        def gather_desc(g, b):
            return pltpu.make_async_copy(
                table.at[idx_v.at[pl.ds(g * _CHUNK, _CHUNK)]],
                rows[b],
                in_sems[b],
            )

        def out_desc(g, b):
            return pltpu.make_async_copy(
                rows[b],
                out.at[pl.ds(base + g * _CHUNK, _CHUNK)],
                out_sems[b],
            )

        def visit(g, b, skip_out_wait=False, skip_gather=False):
            bn = (b + _LOOKAHEAD) % _NBUF
            if not skip_out_wait:
                out_desc(g + _LOOKAHEAD - _NBUF, bn).wait()
            if not skip_gather:
                gather_desc(g + _LOOKAHEAD, bn).start()
            gather_desc(g, b).wait()
            out_desc(g, b).start()

        for g in range(_LOOKAHEAD):
            gather_desc(g, g).start()
        for b in range(_NBUF):
            visit(b, b, skip_out_wait=(b + _LOOKAHEAD < _NBUF))

        def body(o, _):
            g0 = o * _NBUF
            for b in range(_NBUF):
                visit(g0 + b, b)
            return 0

        lax.fori_loop(1, n_chunks // _NBUF - 1, body, 0)

        g0 = n_chunks - _NBUF
        for b in range(_NBUF):
            visit(g0 + b, b, skip_gather=(b >= _NBUF - _LOOKAHEAD))
        for b in range(_NBUF - _LOOKAHEAD, _NBUF):
            out_desc(g0 + b, b).wait()

    return gather_kernel


def kernel(input_ids, table):
    b, s = input_ids.shape
    _, hidden = table.shape
    ids_t = input_ids.T.reshape(-1).astype(jnp.int32)
    return _make_gather(s, b, hidden)(table, ids_t)
